# baseline (speedup 1.0000x reference)
import jax
import jax.numpy as jnp
from jax.experimental import pallas as pl
from jax.experimental.pallas import tpu as pltpu

DEPTH = 4
B_TOK = 16384
CODE_DIM = 256
N_CODES = 1024

BLK = 1024
K_AUG = 384


def _vq_body(z_ref, cb_ref, ct_ref, zq_ref, m0_ref, m1_ref, m2_ref, m3_ref,
             lhs_ref, rhs_ref):
    @pl.when(pl.program_id(0) == 0)
    def _build_aug():
        ct = ct_ref[...]
        ctn = jnp.sum(ct * ct, axis=1, keepdims=True)
        rhs_ref[:, 0:CODE_DIM] = ct * -2.0
        lane = jax.lax.broadcasted_iota(
            jnp.int32, (N_CODES, K_AUG - CODE_DIM), 1)
        rhs_ref[:, CODE_DIM:K_AUG] = jnp.where(lane == 0, ctn, 0.0)
        lane_b = jax.lax.broadcasted_iota(
            jnp.int32, (BLK, K_AUG - CODE_DIM), 1)
        lhs_ref[:, CODE_DIM:K_AUG] = jnp.where(lane_b == 0, 1.0, 0.0)

    r = z_ref[...]
    cb = cb_ref[...]
    maps_refs = (m0_ref, m1_ref, m2_ref, m3_ref)
    iota = jax.lax.broadcasted_iota(jnp.int32, (BLK, N_CODES), 1)
    for i in range(DEPTH):
        lhs_ref[:, 0:CODE_DIM] = r
        dist0 = jax.lax.dot_general(
            lhs_ref[...], rhs_ref[...], (((1,), (1,)), ((), ())),
            preferred_element_type=jnp.float32)
        rn = jnp.sum(r * r, axis=1, keepdims=True)
        maps_refs[i][...] = dist0 + rn
        pred = jnp.argmin(dist0, axis=1)
        onehot = (iota == pred[:, None]).astype(jnp.float32)
        delta = jax.lax.dot_general(
            onehot, cb, (((1,), (0,)), ((), ())),
            preferred_element_type=jnp.float32)
        r = r - delta
    zq_ref[...] = z_ref[...] - r


@jax.jit
def kernel(z, codebook, codebook_t):
    grid = (B_TOK // BLK,)
    row_block = pl.BlockSpec((BLK, CODE_DIM), lambda i: (i, 0))
    full_cb = pl.BlockSpec((N_CODES, CODE_DIM), lambda i: (0, 0))
    map_block = pl.BlockSpec((BLK, N_CODES), lambda i: (i, 0))
    out_shapes = (
        jax.ShapeDtypeStruct((B_TOK, CODE_DIM), jnp.float32),
        *(jax.ShapeDtypeStruct((B_TOK, N_CODES), jnp.float32),) * DEPTH,
    )
    zq, m0, m1, m2, m3 = pl.pallas_call(
        _vq_body,
        grid=grid,
        in_specs=[row_block, full_cb, full_cb],
        out_specs=(row_block, *(map_block,) * DEPTH),
        out_shape=out_shapes,
        scratch_shapes=[
            pltpu.VMEM((BLK, K_AUG), jnp.float32),
            pltpu.VMEM((N_CODES, K_AUG), jnp.float32),
        ],
        compiler_params=pltpu.CompilerParams(
            dimension_semantics=("parallel",)),
    )(z, codebook, codebook_t)
    return (zq, m0, m1, m2, m3)


# speedup vs baseline: 1.0049x; 1.0049x over previous
import jax
import jax.numpy as jnp
from jax.experimental import pallas as pl
from jax.experimental.pallas import tpu as pltpu

DEPTH = 4
B_TOK = 16384
CODE_DIM = 256
N_CODES = 1024

BLK = 1024
K_AUG = 384


def _vq_body(z_ref, cb_ref, ct_ref, zq_ref, m0_ref, m1_ref, m2_ref, m3_ref,
             lhs_ref, rhs_ref):
    ct = ct_ref[...]
    ctn = jnp.sum(ct * ct, axis=1, keepdims=True)
    rhs_ref[:, 0:CODE_DIM] = ct * -2.0
    lane = jax.lax.broadcasted_iota(
        jnp.int32, (N_CODES, K_AUG - CODE_DIM), 1)
    rhs_ref[:, CODE_DIM:K_AUG] = jnp.where(lane == 0, ctn, 0.0)
    lane_b = jax.lax.broadcasted_iota(
        jnp.int32, (BLK, K_AUG - CODE_DIM), 1)
    lhs_ref[:, CODE_DIM:K_AUG] = jnp.where(lane_b == 0, 1.0, 0.0)

    r = z_ref[...]
    cb = cb_ref[...]
    maps_refs = (m0_ref, m1_ref, m2_ref, m3_ref)
    iota = jax.lax.broadcasted_iota(jnp.int32, (BLK, N_CODES), 1)
    for i in range(DEPTH):
        lhs_ref[:, 0:CODE_DIM] = r
        dist0 = jax.lax.dot_general(
            lhs_ref[...], rhs_ref[...], (((1,), (1,)), ((), ())),
            preferred_element_type=jnp.float32)
        rn = jnp.sum(r * r, axis=1, keepdims=True)
        maps_refs[i][...] = dist0 + rn
        pred = jnp.argmin(dist0, axis=1)
        onehot = (iota == pred[:, None]).astype(jnp.float32)
        delta = jax.lax.dot_general(
            onehot, cb, (((1,), (0,)), ((), ())),
            preferred_element_type=jnp.float32)
        r = r - delta
    zq_ref[...] = z_ref[...] - r


@jax.jit
def kernel(z, codebook, codebook_t):
    grid = (B_TOK // BLK,)
    row_block = pl.BlockSpec((BLK, CODE_DIM), lambda i: (i, 0))
    full_cb = pl.BlockSpec((N_CODES, CODE_DIM), lambda i: (0, 0))
    map_block = pl.BlockSpec((BLK, N_CODES), lambda i: (i, 0))
    out_shapes = (
        jax.ShapeDtypeStruct((B_TOK, CODE_DIM), jnp.float32),
        *(jax.ShapeDtypeStruct((B_TOK, N_CODES), jnp.float32),) * DEPTH,
    )
    zq, m0, m1, m2, m3 = pl.pallas_call(
        _vq_body,
        grid=grid,
        in_specs=[row_block, full_cb, full_cb],
        out_specs=(row_block, *(map_block,) * DEPTH),
        out_shape=out_shapes,
        scratch_shapes=[
            pltpu.VMEM((BLK, K_AUG), jnp.float32),
            pltpu.VMEM((N_CODES, K_AUG), jnp.float32),
        ],
        compiler_params=pltpu.CompilerParams(
            dimension_semantics=("parallel",)),
    )(z, codebook, codebook_t)
    return (zq, m0, m1, m2, m3)


# speedup vs baseline: 1.5905x; 1.5828x over previous
import jax
import jax.numpy as jnp
from jax.experimental import pallas as pl
from jax.experimental.pallas import tpu as pltpu

DEPTH = 4
B_TOK = 16384
CODE_DIM = 256
N_CODES = 1024

BLK = 1024
HALF = BLK // 2


def _vq_body(z_ref, cb_ref, ct_ref, zq_ref, m0_ref, m1_ref, m2_ref, m3_ref):
    ct = ct_ref[...]
    cb = cb_ref[...]
    ctm = ct * -2.0
    ctn = jnp.sum(ct * ct, axis=1)
    maps_refs = (m0_ref, m1_ref, m2_ref, m3_ref)
    iota = jax.lax.broadcasted_iota(jnp.int32, (HALF, N_CODES), 1)
    rs = [z_ref[0:HALF, :], z_ref[HALF:BLK, :]]
    for i in range(DEPTH):
        prods = [jax.lax.dot_general(
            r, ctm, (((1,), (1,)), ((), ())),
            preferred_element_type=jnp.float32) for r in rs]
        for h in range(2):
            r = rs[h]
            rn = jnp.sum(r * r, axis=1, keepdims=True)
            dist = (prods[h] + ctn[None, :]) + rn
            maps_refs[i][h * HALF:(h + 1) * HALF, :] = dist
            pred = jnp.argmin(dist, axis=1)
            onehot = (iota == pred[:, None]).astype(jnp.float32)
            delta = jax.lax.dot_general(
                onehot, cb, (((1,), (0,)), ((), ())),
                preferred_element_type=jnp.float32)
            rs[h] = r - delta
    zq_ref[0:HALF, :] = z_ref[0:HALF, :] - rs[0]
    zq_ref[HALF:BLK, :] = z_ref[HALF:BLK, :] - rs[1]


@jax.jit
def kernel(z, codebook, codebook_t):
    grid = (B_TOK // BLK,)
    row_block = pl.BlockSpec((BLK, CODE_DIM), lambda i: (i, 0))
    full_cb = pl.BlockSpec((N_CODES, CODE_DIM), lambda i: (0, 0))
    map_block = pl.BlockSpec((BLK, N_CODES), lambda i: (i, 0))
    out_shapes = (
        jax.ShapeDtypeStruct((B_TOK, CODE_DIM), jnp.float32),
        *(jax.ShapeDtypeStruct((B_TOK, N_CODES), jnp.float32),) * DEPTH,
    )
    zq, m0, m1, m2, m3 = pl.pallas_call(
        _vq_body,
        grid=grid,
        in_specs=[row_block, full_cb, full_cb],
        out_specs=(row_block, *(map_block,) * DEPTH),
        out_shape=out_shapes,
        compiler_params=pltpu.CompilerParams(
            dimension_semantics=("parallel",)),
    )(z, codebook, codebook_t)
    return (zq, m0, m1, m2, m3)
